# baseline (speedup 1.0000x reference)
import jax
import jax.numpy as jnp
from jax import lax
from jax.experimental import pallas as pl
from jax.experimental.pallas import tpu as pltpu
from jax.experimental.pallas import tpu_sc as plsc

_K = 32
_L = 16


def _dot_body(iu_hbm, ip_hbm, u_hbm, p_hbm, out_hbm,
              idx_u, idx_p, u_row, p_row, out_v, sem):
    is_lead = (lax.axis_index("c") == 0) & (lax.axis_index("s") == 0)

    @pl.when(is_lead)
    def _():
        pltpu.sync_copy(iu_hbm, idx_u)
        pltpu.sync_copy(ip_hbm, idx_p)
        cu = pltpu.async_copy(u_hbm.at[idx_u], u_row, sem)
        cp = pltpu.async_copy(p_hbm.at[idx_p], p_row, sem)
        cu.wait()
        cp.wait()
        u0 = u_row[0, pl.ds(0, _L)]
        u1 = u_row[0, pl.ds(_L, _L)]
        p0 = p_row[0, pl.ds(0, _L)]
        p1 = p_row[0, pl.ds(_L, _L)]
        prod = u0 * p0 + u1 * p1
        out_v[...] = jnp.zeros((_L,), jnp.float32)
        plsc.addupdate_scatter(out_v, [jnp.zeros((_L,), jnp.int32)], prod)
        pltpu.sync_copy(out_v, out_hbm)


_sc_dot = pl.kernel(
    _dot_body,
    out_type=jax.ShapeDtypeStruct((_L,), jnp.float32),
    mesh=plsc.VectorSubcoreMesh(core_axis_name="c", subcore_axis_name="s"),
    compiler_params=pltpu.CompilerParams(
        needs_layout_passes=False, use_tc_tiling_on_sc=False),
    scratch_types=[
        pltpu.VMEM((1,), jnp.int32),
        pltpu.VMEM((1,), jnp.int32),
        pltpu.VMEM((1, _K), jnp.float32),
        pltpu.VMEM((1, _K), jnp.float32),
        pltpu.VMEM((_L,), jnp.float32),
        pltpu.SemaphoreType.DMA,
    ],
)


def kernel(i_user, i_product, U, P):
    iu = jnp.asarray(i_user, jnp.int32).reshape(1)
    ip = jnp.asarray(i_product, jnp.int32).reshape(1)
    out = _sc_dot(iu, ip, U, P)
    return out[0]


# speedup vs baseline: 1.4673x; 1.4673x over previous
import jax
import jax.numpy as jnp
from jax import lax
from jax.experimental import pallas as pl
from jax.experimental.pallas import tpu as pltpu
from jax.experimental.pallas import tpu_sc as plsc

_K = 32
_L = 16


def _dot_body(iu_hbm, ip_hbm, u_hbm, p_hbm, out_hbm,
              idx_u, idx_p, u_tile, p_tile, out_v, sem):
    is_lead = (lax.axis_index("c") == 0) & (lax.axis_index("s") == 0)

    @pl.when(is_lead)
    def _():
        pltpu.sync_copy(iu_hbm, idx_u)
        pltpu.sync_copy(ip_hbm, idx_p)
        iu = idx_u[...][0]
        ip = idx_p[...][0]
        bu = pl.multiple_of((iu // 8) * 8, 8)
        bp = pl.multiple_of((ip // 8) * 8, 8)
        cu = pltpu.async_copy(u_hbm.at[pl.ds(bu, 8), :], u_tile, sem)
        cp = pltpu.async_copy(p_hbm.at[pl.ds(bp, 8), :], p_tile, sem)
        cu.wait()
        cp.wait()
        lanes = lax.iota(jnp.int32, _L)
        ru = jnp.broadcast_to(iu % 8, (_L,))
        rp = jnp.broadcast_to(ip % 8, (_L,))
        u0 = plsc.load_gather(u_tile, [ru, lanes])
        u1 = plsc.load_gather(u_tile, [ru, lanes + _L])
        p0 = plsc.load_gather(p_tile, [rp, lanes])
        p1 = plsc.load_gather(p_tile, [rp, lanes + _L])
        prod = u0 * p0 + u1 * p1
        out_v[...] = jnp.zeros((_L,), jnp.float32)
        plsc.addupdate_scatter(out_v, [jnp.zeros((_L,), jnp.int32)], prod)
        pltpu.sync_copy(out_v, out_hbm)


_sc_dot = pl.kernel(
    _dot_body,
    out_type=jax.ShapeDtypeStruct((_L,), jnp.float32),
    mesh=plsc.VectorSubcoreMesh(core_axis_name="c", subcore_axis_name="s"),
    compiler_params=pltpu.CompilerParams(needs_layout_passes=False),
    scratch_types=[
        pltpu.VMEM((16,), jnp.int32),
        pltpu.VMEM((16,), jnp.int32),
        pltpu.VMEM((8, _K), jnp.float32),
        pltpu.VMEM((8, _K), jnp.float32),
        pltpu.VMEM((_L,), jnp.float32),
        pltpu.SemaphoreType.DMA,
    ],
)


def kernel(i_user, i_product, U, P):
    iu = jnp.full((16,), i_user, jnp.int32)
    ip = jnp.full((16,), i_product, jnp.int32)
    out = _sc_dot(iu, ip, U, P)
    return out[0]


# speedup vs baseline: 1.4737x; 1.0044x over previous
import jax
import jax.numpy as jnp
from jax import lax
from jax.experimental import pallas as pl
from jax.experimental.pallas import tpu as pltpu
from jax.experimental.pallas import tpu_sc as plsc

_K = 32
_L = 16


def _dot_body(iu_hbm, ip_hbm, u_hbm, p_hbm, out_hbm,
              idx_u, idx_p, u_tile, p_tile, out_v, sem):
    is_lead = (lax.axis_index("c") == 0) & (lax.axis_index("s") == 0)

    @pl.when(is_lead)
    def _():
        pltpu.sync_copy(iu_hbm, idx_u)
        pltpu.sync_copy(ip_hbm, idx_p)
        iu = idx_u[...][0]
        ip = idx_p[...][0]
        bu = pl.multiple_of((iu // 8) * 8, 8)
        bp = pl.multiple_of((ip // 8) * 8, 8)
        cu = pltpu.async_copy(u_hbm.at[pl.ds(bu, 8), :], u_tile, sem)
        cp = pltpu.async_copy(p_hbm.at[pl.ds(bp, 8), :], p_tile, sem)
        cu.wait()
        cp.wait()
        lanes = lax.iota(jnp.int32, _L)
        ru = jnp.broadcast_to(iu % 8, (_L,))
        rp = jnp.broadcast_to(ip % 8, (_L,))
        u0 = plsc.load_gather(u_tile, [ru, lanes])
        u1 = plsc.load_gather(u_tile, [ru, lanes + _L])
        p0 = plsc.load_gather(p_tile, [rp, lanes])
        p1 = plsc.load_gather(p_tile, [rp, lanes + _L])
        prod = u0 * p0 + u1 * p1
        out_v[...] = jnp.zeros((_L,), jnp.float32)
        plsc.addupdate_scatter(out_v, [jnp.zeros((_L,), jnp.int32)], prod)
        pltpu.sync_copy(out_v, out_hbm)


_sc_dot = pl.kernel(
    _dot_body,
    out_type=jax.ShapeDtypeStruct((_L,), jnp.float32),
    mesh=plsc.VectorSubcoreMesh(core_axis_name="c", subcore_axis_name="s"),
    compiler_params=pltpu.CompilerParams(
        needs_layout_passes=False, use_tc_tiling_on_sc=True),
    scratch_types=[
        pltpu.VMEM((16,), jnp.int32),
        pltpu.VMEM((16,), jnp.int32),
        pltpu.VMEM((8, _K), jnp.float32),
        pltpu.VMEM((8, _K), jnp.float32),
        pltpu.VMEM((_L,), jnp.float32),
        pltpu.SemaphoreType.DMA,
    ],
)


def kernel(i_user, i_product, U, P):
    iu = jnp.full((16,), i_user, jnp.int32)
    ip = jnp.full((16,), i_product, jnp.int32)
    out = _sc_dot(iu, ip, U, P)
    return out[0]


# speedup vs baseline: 40.8850x; 27.7436x over previous
import jax
import jax.numpy as jnp
from jax import lax
from jax.experimental import pallas as pl
from jax.experimental.pallas import tpu as pltpu
from jax.experimental.pallas import tpu_sc as plsc

_K = 32
_L = 16
_B = 128


def _dot_body(iu_hbm, ip_hbm, u_hbm, p_hbm, out_hbm,
              idx_u, idx_p, u_blk, p_blk, out_v, sem):
    is_lead = (lax.axis_index("c") == 0) & (lax.axis_index("s") == 0)

    @pl.when(is_lead)
    def _():
        pltpu.sync_copy(iu_hbm, idx_u)
        pltpu.sync_copy(ip_hbm, idx_p)
        iu = idx_u[...][0]
        ip = idx_p[...][0]
        bu = pl.multiple_of((iu // _B) * _B, _B)
        bp = pl.multiple_of((ip // _B) * _B, _B)
        cu = pltpu.async_copy(u_hbm.at[:, pl.ds(bu, _B)], u_blk, sem)
        cp = pltpu.async_copy(p_hbm.at[:, pl.ds(bp, _B)], p_blk, sem)
        cu.wait()
        cp.wait()
        lanes = lax.iota(jnp.int32, _L)
        cu_idx = jnp.broadcast_to(iu % _B, (_L,))
        cp_idx = jnp.broadcast_to(ip % _B, (_L,))
        u0 = plsc.load_gather(u_blk, [lanes, cu_idx])
        u1 = plsc.load_gather(u_blk, [lanes + _L, cu_idx])
        p0 = plsc.load_gather(p_blk, [lanes, cp_idx])
        p1 = plsc.load_gather(p_blk, [lanes + _L, cp_idx])
        prod = u0 * p0 + u1 * p1
        out_v[...] = jnp.zeros((_L,), jnp.float32)
        plsc.addupdate_scatter(out_v, [jnp.zeros((_L,), jnp.int32)], prod)
        pltpu.sync_copy(out_v, out_hbm)


_sc_dot = pl.kernel(
    _dot_body,
    out_type=jax.ShapeDtypeStruct((_L,), jnp.float32),
    mesh=plsc.VectorSubcoreMesh(core_axis_name="c", subcore_axis_name="s"),
    compiler_params=pltpu.CompilerParams(
        needs_layout_passes=False, use_tc_tiling_on_sc=True),
    scratch_types=[
        pltpu.VMEM((16,), jnp.int32),
        pltpu.VMEM((16,), jnp.int32),
        pltpu.VMEM((_K, _B), jnp.float32),
        pltpu.VMEM((_K, _B), jnp.float32),
        pltpu.VMEM((_L,), jnp.float32),
        pltpu.SemaphoreType.DMA,
    ],
)


def kernel(i_user, i_product, U, P):
    iu = jnp.full((16,), i_user, jnp.int32)
    ip = jnp.full((16,), i_product, jnp.int32)
    out = _sc_dot(iu, ip, U.T, P.T)
    return out[0]


# speedup vs baseline: 45.5670x; 1.1145x over previous
import jax
import jax.numpy as jnp
from jax import lax
from jax.experimental import pallas as pl
from jax.experimental.pallas import tpu as pltpu
from jax.experimental.pallas import tpu_sc as plsc

_K = 32
_L = 16
_B = 128


def _dot_body(idx_hbm, u_hbm, p_hbm, out_hbm,
              idx_v, u_blk, p_blk, out_v, sem):
    is_lead = lax.axis_index("s") == 0

    @pl.when(is_lead)
    def _():
        pltpu.sync_copy(idx_hbm, idx_v)
        iv = idx_v[...]
        iu = iv[0]
        ip = iv[1]
        bu = pl.multiple_of((iu // _B) * _B, _B)
        bp = pl.multiple_of((ip // _B) * _B, _B)
        cu = pltpu.async_copy(u_hbm.at[:, pl.ds(bu, _B)], u_blk, sem)
        cp = pltpu.async_copy(p_hbm.at[:, pl.ds(bp, _B)], p_blk, sem)
        cu.wait()
        cp.wait()
        lanes = lax.iota(jnp.int32, _L)
        cu_idx = jnp.broadcast_to(iu % _B, (_L,))
        cp_idx = jnp.broadcast_to(ip % _B, (_L,))
        u0 = plsc.load_gather(u_blk, [lanes, cu_idx])
        u1 = plsc.load_gather(u_blk, [lanes + _L, cu_idx])
        p0 = plsc.load_gather(p_blk, [lanes, cp_idx])
        p1 = plsc.load_gather(p_blk, [lanes + _L, cp_idx])
        prod = u0 * p0 + u1 * p1
        out_v[...] = jnp.zeros((_L,), jnp.float32)
        plsc.addupdate_scatter(out_v, [jnp.zeros((_L,), jnp.int32)], prod)
        pltpu.sync_copy(out_v, out_hbm)


_sc_dot = pl.kernel(
    _dot_body,
    out_type=jax.ShapeDtypeStruct((_L,), jnp.float32),
    mesh=plsc.VectorSubcoreMesh(
        core_axis_name="c", subcore_axis_name="s", num_cores=1),
    compiler_params=pltpu.CompilerParams(
        needs_layout_passes=False, use_tc_tiling_on_sc=True),
    scratch_types=[
        pltpu.VMEM((_L,), jnp.int32),
        pltpu.VMEM((_K, _B), jnp.float32),
        pltpu.VMEM((_K, _B), jnp.float32),
        pltpu.VMEM((_L,), jnp.float32),
        pltpu.SemaphoreType.DMA,
    ],
)


def kernel(i_user, i_product, U, P):
    idx = jnp.zeros((_L,), jnp.int32)
    idx = idx.at[0].set(i_user).at[1].set(i_product)
    out = _sc_dot(idx, U.T, P.T)
    return out[0]
